# Initial kernel scaffold; baseline (speedup 1.0000x reference)
import jax
import jax.numpy as jnp
from jax.experimental import pallas as pl


def kernel(x, edge_index, W0, b0, W1, b1, WF, bF):
    raise NotImplementedError("write your pallas kernel here")


# speedup vs baseline: 15.8499x; 15.8499x over previous
import functools

import jax
import jax.numpy as jnp
from jax import lax
from jax.experimental import pallas as pl
from jax.experimental.pallas import tpu as pltpu
from jax.experimental.pallas import tpu_sc as plsc

N = 10000
E = 320000
D = 128
NC, NS = 2, 16
NW = NC * NS
EPT = E // NW
K = 128
FULL = EPT // K
TAIL = EPT - FULL * K
R0 = 632
R1 = N - (NS - 1) * R0

_mesh = plsc.VectorSubcoreMesh(core_axis_name="c", subcore_axis_name="s")


def _part_copy(src_ref, dst_ref, s, src_off, dst_off):
    r0 = pl.multiple_of(s * R0, 8)

    @pl.when(s < NS - 1)
    def _():
        pltpu.sync_copy(src_ref.at[pl.ds(pl.multiple_of(src_off + r0, 8), R0)],
                        dst_ref.at[pl.ds(pl.multiple_of(dst_off + r0, 8), R0)])

    @pl.when(s == NS - 1)
    def _():
        last = (NS - 1) * R0
        pltpu.sync_copy(src_ref.at[pl.ds(pl.multiple_of(src_off + last, 8), R1)],
                        dst_ref.at[pl.ds(pl.multiple_of(dst_off + last, 8), R1)])


@functools.partial(
    pl.kernel,
    mesh=_mesh,
    out_type=jax.ShapeDtypeStruct((NC * N, 16), jnp.float32),
    scratch_types=[
        pltpu.VMEM((K, 16), jnp.float32),
        pltpu.VMEM((K,), jnp.int32),
        pltpu.VMEM((TAIL,), jnp.int32),
        pltpu.VMEM_SHARED((N, 16), jnp.float32),
    ],
)
def _deg(dst_hbm, ones_hbm, zeros16_hbm, degw_out, ones_v, idx_v, tidx_v, deg_sh):
    c = lax.axis_index("c")
    s = lax.axis_index("s")
    wid = s * NC + c
    base = wid * EPT
    _part_copy(zeros16_hbm, deg_sh, s, 0, 0)
    pltpu.sync_copy(ones_hbm, ones_v)
    plsc.subcore_barrier()

    def body(g, carry):
        pltpu.sync_copy(dst_hbm.at[pl.ds(base + g * K, K)], idx_v)
        pltpu.sync_copy(ones_v, deg_sh.at[idx_v], add=True)
        return carry

    lax.fori_loop(0, FULL, body, 0)
    pltpu.sync_copy(dst_hbm.at[pl.ds(base + FULL * K, TAIL)], tidx_v)
    pltpu.sync_copy(ones_v.at[pl.ds(0, TAIL)], deg_sh.at[tidx_v], add=True)
    plsc.subcore_barrier()
    _part_copy(deg_sh, degw_out, s, 0, c * N)


@functools.partial(
    pl.kernel,
    mesh=_mesh,
    out_type=jax.ShapeDtypeStruct((NC * N, D), jnp.float32),
    scratch_types=[
        pltpu.VMEM((K,), jnp.int32),
        pltpu.VMEM((K,), jnp.int32),
        pltpu.VMEM((K, D), jnp.float32),
        pltpu.VMEM((TAIL,), jnp.int32),
        pltpu.VMEM((TAIL,), jnp.int32),
        pltpu.VMEM((TAIL, D), jnp.float32),
        pltpu.VMEM_SHARED((N, D), jnp.float32),
        pltpu.SemaphoreType.DMA,
    ],
)
def _agg(y_hbm, src_hbm, dst_hbm, zeros_hbm, z_out,
         src_v, dst_v, rows_v, tsrc_v, tdst_v, trows_v, z_sh, sem):
    c = lax.axis_index("c")
    s = lax.axis_index("s")
    wid = s * NC + c
    base = wid * EPT
    _part_copy(zeros_hbm, z_sh, s, 0, 0)
    plsc.subcore_barrier()

    def body(g, carry):
        off = base + g * K
        pltpu.sync_copy(src_hbm.at[pl.ds(off, K)], src_v)
        cp = pltpu.async_copy(y_hbm.at[src_v], rows_v, sem)
        pltpu.sync_copy(dst_hbm.at[pl.ds(off, K)], dst_v)
        cp.wait()
        pltpu.sync_copy(rows_v, z_sh.at[dst_v], add=True)
        return carry

    lax.fori_loop(0, FULL, body, 0)
    toff = base + FULL * K
    pltpu.sync_copy(src_hbm.at[pl.ds(toff, TAIL)], tsrc_v)
    pltpu.sync_copy(dst_hbm.at[pl.ds(toff, TAIL)], tdst_v)
    pltpu.async_copy(y_hbm.at[tsrc_v], trows_v, sem).wait()
    pltpu.sync_copy(trows_v, z_sh.at[tdst_v], add=True)
    plsc.subcore_barrier()
    _part_copy(z_sh, z_out, s, 0, c * N)


B = 1000
GRID = N // B


def _pre_body(x_ref, w_ref, d0_ref, d1_ref, y_ref, dv_ref):
    deg = d0_ref[:, 0:1] + d1_ref[:, 0:1] + 1.0
    dv = jnp.broadcast_to(lax.rsqrt(deg), (B, D))
    dv_ref[...] = dv
    y_ref[...] = jnp.dot(x_ref[...], w_ref[...],
                         preferred_element_type=jnp.float32) * dv


def _mid_body(z0_ref, z1_ref, y_ref, dv_ref, b_ref, w_ref, o_ref):
    dv = dv_ref[...]
    agg = (z0_ref[...] + z1_ref[...] + y_ref[...]) * dv + b_ref[...]
    h = jnp.maximum(agg, 0.0)
    o_ref[...] = jnp.dot(h, w_ref[...], preferred_element_type=jnp.float32) * dv


def _fin_body(z0_ref, z1_ref, y_ref, dv_ref, b_ref, o_ref):
    o_ref[...] = ((z0_ref[...] + z1_ref[...] + y_ref[...]) * dv_ref[...]
                  + b_ref[...])


_row = pl.BlockSpec((B, D), lambda i: (i, 0))
_row0 = pl.BlockSpec((B, D), lambda i: (i, 0))
_row1 = pl.BlockSpec((B, D), lambda i: (i + N // B, 0))
_w = pl.BlockSpec((D, D), lambda i: (0, 0))
_bvec = pl.BlockSpec((1, D), lambda i: (0, 0))
_d0 = pl.BlockSpec((B, 16), lambda i: (i, 0))
_d1 = pl.BlockSpec((B, 16), lambda i: (i + N // B, 0))

_pre = pl.pallas_call(
    _pre_body, grid=(GRID,),
    in_specs=[_row, _w, _d0, _d1],
    out_specs=[_row, _row],
    out_shape=[jax.ShapeDtypeStruct((N, D), jnp.float32),
               jax.ShapeDtypeStruct((N, D), jnp.float32)],
)

_mid = pl.pallas_call(
    _mid_body, grid=(GRID,),
    in_specs=[_row0, _row1, _row, _row, _bvec, _w],
    out_specs=_row,
    out_shape=jax.ShapeDtypeStruct((N, D), jnp.float32),
)

_fin = pl.pallas_call(
    _fin_body, grid=(GRID,),
    in_specs=[_row0, _row1, _row, _row, _bvec],
    out_specs=_row,
    out_shape=jax.ShapeDtypeStruct((N, D), jnp.float32),
)


def kernel(x, edge_index, W0, b0, W1, b1, WF, bF):
    src = edge_index[0].astype(jnp.int32)
    dst = edge_index[1].astype(jnp.int32)
    ones16 = jnp.ones((K, 16), jnp.float32)
    zeros16 = jnp.zeros((N, 16), jnp.float32)
    zeros = jnp.zeros((N, D), jnp.float32)

    degw = _deg(dst, ones16, zeros16)
    y0, dv = _pre(x, W0, degw, degw)
    zz = _agg(y0, src, dst, zeros)
    y1 = _mid(zz, zz, y0, dv, b0.reshape(1, D), W1)
    zz = _agg(y1, src, dst, zeros)
    y2 = _mid(zz, zz, y1, dv, b1.reshape(1, D), WF)
    zz = _agg(y2, src, dst, zeros)
    return _fin(zz, zz, y2, dv, bF.reshape(1, D))
